# baseline (speedup 1.0000x reference)
import jax
import jax.numpy as jnp
from jax.experimental import pallas as pl
from jax.experimental.pallas import tpu as pltpu

_N = 4096
_D = 128
_R = 1024
_BIG = 1e10
_TARGET_RATE = 1.0
_BETA = 0.1
_PSI_K = 0.9227843350984671
_PSI_N = 8.317644091471843


def _digamma_ge1(t):
    s = 1.0 / t + 1.0 / (t + 1.0)
    u = t + 2.0
    w = 1.0 / (u * u)
    series = jnp.log(u) - 0.5 / u - w * (
        1.0 / 12.0 - w * (1.0 / 120.0 - w * (1.0 / 252.0)))
    return series - s


def _ccl_kernel(xr_ref, yr_ref, x_ref, y_ref,
                tl_ref, mi_ref, rl_ref, cl_ref,
                acc_ref, sx_ref, sy_ref, sqx_ref, sqy_ref, xb_ref, yb_ref):
    i = pl.program_id(0)
    nsteps = pl.num_programs(0)
    dn = (((1,), (1,)), ((), ()))

    xr = xr_ref[...]
    yr = yr_ref[...]

    @pl.when(i == 0)
    def _init():
        xf = x_ref[...]
        yf = y_ref[...]
        sqx_ref[...] = jnp.sum(xf * xf, axis=1, keepdims=True)
        sqy_ref[...] = jnp.sum(yf * yf, axis=1, keepdims=True)
        xb_ref[...] = xf.astype(jnp.bfloat16)
        yb_ref[...] = yf.astype(jnp.bfloat16)
        acc_ref[...] = jnp.zeros_like(acc_ref)
        sx_ref[...] = jnp.zeros_like(sx_ref)
        sy_ref[...] = jnp.zeros_like(sy_ref)

    ones = jnp.ones((1, _D), jnp.float32)
    sq_xr = jax.lax.dot_general(ones, xr * xr, dn,
                                preferred_element_type=jnp.float32)
    sq_yr = jax.lax.dot_general(ones, yr * yr, dn,
                                preferred_element_type=jnp.float32)

    gx = jax.lax.dot_general(xb_ref[...], (-2.0 * xr).astype(jnp.bfloat16),
                             dn, preferred_element_type=jnp.float32)
    gy = jax.lax.dot_general(yb_ref[...], (-2.0 * yr).astype(jnp.bfloat16),
                             dn, preferred_element_type=jnp.float32)
    ax = gx + sqx_ref[...]
    ay = gy + sqy_ref[...]

    dz = ax + ay

    m1 = jnp.min(dz, axis=0, keepdims=True)
    m2 = jnp.min(jnp.where(dz > m1, dz, _BIG), axis=0, keepdims=True)
    m3 = jnp.min(jnp.where(dz > m2, dz, _BIG), axis=0, keepdims=True)
    eps = jnp.min(jnp.where(dz > m3, dz, _BIG), axis=0, keepdims=True)

    tx = eps + sq_yr
    ty = eps + sq_xr
    nx = jnp.sum((ax < tx).astype(jnp.float32), axis=0, keepdims=True)
    ny = jnp.sum((ay < ty).astype(jnp.float32), axis=0, keepdims=True)
    part = jnp.sum(_digamma_ge1(nx) + _digamma_ge1(ny))

    acc_ref[...] += jnp.reshape(part, (1, 1))
    sx_ref[...] += jnp.sum(xr, axis=0, keepdims=True)
    sy_ref[...] += jnp.sum(yr, axis=0, keepdims=True)

    @pl.when(i == nsteps - 1)
    def _finalize():
        inv_n = 1.0 / _N
        mi = _PSI_K + _PSI_N - jnp.sum(acc_ref[...]) * inv_n
        p_in = sx_ref[...] * inv_n
        p_out = sy_ref[...] * inv_n
        h_in = -jnp.sum(p_in * jnp.log(p_in + 1e-10))
        h_out = -jnp.sum(p_out * jnp.log(p_out + 1e-10))
        rate_loss = jnp.abs(mi - _TARGET_RATE)
        cap = -mi + _BETA * (h_in + h_out)
        mi_ref[...] = jnp.reshape(mi, (1, 1))
        rl_ref[...] = jnp.reshape(rate_loss, (1, 1))
        cl_ref[...] = jnp.reshape(cap, (1, 1))
        tl_ref[...] = jnp.reshape(rate_loss + cap, (1, 1))


def kernel(inputs, outputs):
    scalar = jax.ShapeDtypeStruct((1, 1), jnp.float32)
    tl, mi, rl, cl = pl.pallas_call(
        _ccl_kernel,
        grid=(_N // _R,),
        in_specs=[
            pl.BlockSpec((_R, _D), lambda i: (i, 0)),
            pl.BlockSpec((_R, _D), lambda i: (i, 0)),
            pl.BlockSpec((_N, _D), lambda i: (0, 0)),
            pl.BlockSpec((_N, _D), lambda i: (0, 0)),
        ],
        out_specs=[pl.BlockSpec((1, 1), lambda i: (0, 0))] * 4,
        out_shape=[scalar] * 4,
        scratch_shapes=[
            pltpu.VMEM((1, 1), jnp.float32),
            pltpu.VMEM((1, _D), jnp.float32),
            pltpu.VMEM((1, _D), jnp.float32),
            pltpu.VMEM((_N, 1), jnp.float32),
            pltpu.VMEM((_N, 1), jnp.float32),
            pltpu.VMEM((_N, _D), jnp.bfloat16),
            pltpu.VMEM((_N, _D), jnp.bfloat16),
        ],
        compiler_params=pltpu.CompilerParams(
            dimension_semantics=("arbitrary",)),
    )(inputs, outputs, inputs, outputs)
    return (tl[0, 0], mi[0, 0], rl[0, 0], cl[0, 0])


# speedup vs baseline: 1.0519x; 1.0519x over previous
import jax
import jax.numpy as jnp
from jax.experimental import pallas as pl
from jax.experimental.pallas import tpu as pltpu

_N = 4096
_D = 128
_R = 1024
_BIG = 1e10
_TARGET_RATE = 1.0
_BETA = 0.1
_PSI_K = 0.9227843350984671
_PSI_N = 8.317644091471843


def _digamma_ge1(t):
    s = 1.0 / t + 1.0 / (t + 1.0)
    u = t + 2.0
    w = 1.0 / (u * u)
    series = jnp.log(u) - 0.5 / u - w * (
        1.0 / 12.0 - w * (1.0 / 120.0 - w * (1.0 / 252.0)))
    return series - s


def _ccl_kernel(xr_ref, yr_ref, x_ref, y_ref,
                tl_ref, mi_ref, rl_ref, cl_ref,
                acc_ref, sx_ref, sy_ref, sqx_ref, sqy_ref, xb_ref, yb_ref):
    i = pl.program_id(0)
    nsteps = pl.num_programs(0)
    dn = (((1,), (1,)), ((), ()))

    xr = xr_ref[...]
    yr = yr_ref[...]

    @pl.when(i == 0)
    def _init():
        xf = x_ref[...]
        yf = y_ref[...]
        sqx_ref[...] = jnp.sum(xf * xf, axis=1, keepdims=True)
        sqy_ref[...] = jnp.sum(yf * yf, axis=1, keepdims=True)
        xb_ref[...] = xf.astype(jnp.bfloat16)
        yb_ref[...] = yf.astype(jnp.bfloat16)
        acc_ref[...] = jnp.zeros_like(acc_ref)
        sx_ref[...] = jnp.zeros_like(sx_ref)
        sy_ref[...] = jnp.zeros_like(sy_ref)

    ones = jnp.ones((1, _D), jnp.float32)
    sq_xr = jax.lax.dot_general(ones, xr * xr, dn,
                                preferred_element_type=jnp.float32)
    sq_yr = jax.lax.dot_general(ones, yr * yr, dn,
                                preferred_element_type=jnp.float32)

    gx = jax.lax.dot_general(xb_ref[...], (-2.0 * xr).astype(jnp.bfloat16),
                             dn, preferred_element_type=jnp.float32)
    gy = jax.lax.dot_general(yb_ref[...], (-2.0 * yr).astype(jnp.bfloat16),
                             dn, preferred_element_type=jnp.float32)
    ax = gx + sqx_ref[...]
    ay = gy + sqy_ref[...]

    rows = jax.lax.broadcasted_iota(jnp.int32, (_N, _R), 0)
    cols = jax.lax.broadcasted_iota(jnp.int32, (_N, _R), 1)
    diag = rows == (i * _R + cols)
    dz = jnp.where(diag, _BIG, ax + ay)

    m1 = jnp.min(dz, axis=0, keepdims=True)
    m2 = jnp.min(jnp.where(dz > m1, dz, _BIG), axis=0, keepdims=True)
    eps = jnp.min(jnp.where(dz > m2, dz, _BIG), axis=0, keepdims=True)

    tx = eps + sq_yr
    ty = eps + sq_xr
    nx = jnp.sum((ax < tx).astype(jnp.float32), axis=0, keepdims=True)
    ny = jnp.sum((ay < ty).astype(jnp.float32), axis=0, keepdims=True)
    part = jnp.sum(_digamma_ge1(nx) + _digamma_ge1(ny))

    acc_ref[...] += jnp.reshape(part, (1, 1))
    sx_ref[...] += jnp.sum(xr, axis=0, keepdims=True)
    sy_ref[...] += jnp.sum(yr, axis=0, keepdims=True)

    @pl.when(i == nsteps - 1)
    def _finalize():
        inv_n = 1.0 / _N
        mi = _PSI_K + _PSI_N - jnp.sum(acc_ref[...]) * inv_n
        p_in = sx_ref[...] * inv_n
        p_out = sy_ref[...] * inv_n
        h_in = -jnp.sum(p_in * jnp.log(p_in + 1e-10))
        h_out = -jnp.sum(p_out * jnp.log(p_out + 1e-10))
        rate_loss = jnp.abs(mi - _TARGET_RATE)
        cap = -mi + _BETA * (h_in + h_out)
        mi_ref[...] = jnp.reshape(mi, (1, 1))
        rl_ref[...] = jnp.reshape(rate_loss, (1, 1))
        cl_ref[...] = jnp.reshape(cap, (1, 1))
        tl_ref[...] = jnp.reshape(rate_loss + cap, (1, 1))


def kernel(inputs, outputs):
    scalar = jax.ShapeDtypeStruct((1, 1), jnp.float32)
    tl, mi, rl, cl = pl.pallas_call(
        _ccl_kernel,
        grid=(_N // _R,),
        in_specs=[
            pl.BlockSpec((_R, _D), lambda i: (i, 0)),
            pl.BlockSpec((_R, _D), lambda i: (i, 0)),
            pl.BlockSpec((_N, _D), lambda i: (0, 0)),
            pl.BlockSpec((_N, _D), lambda i: (0, 0)),
        ],
        out_specs=[pl.BlockSpec((1, 1), lambda i: (0, 0))] * 4,
        out_shape=[scalar] * 4,
        scratch_shapes=[
            pltpu.VMEM((1, 1), jnp.float32),
            pltpu.VMEM((1, _D), jnp.float32),
            pltpu.VMEM((1, _D), jnp.float32),
            pltpu.VMEM((_N, 1), jnp.float32),
            pltpu.VMEM((_N, 1), jnp.float32),
            pltpu.VMEM((_N, _D), jnp.bfloat16),
            pltpu.VMEM((_N, _D), jnp.bfloat16),
        ],
        compiler_params=pltpu.CompilerParams(
            dimension_semantics=("arbitrary",)),
    )(inputs, outputs, inputs, outputs)
    return (tl[0, 0], mi[0, 0], rl[0, 0], cl[0, 0])
